# baseline (speedup 1.0000x reference)
import jax
import jax.numpy as jnp
from jax import lax
from jax.experimental import pallas as pl
from jax.experimental.pallas import tpu as pltpu
from jax.experimental.pallas import tpu_sc as plsc

_B = 1024
_S = 200
_D = 64
_V = 1000000
_NC = 2
_NS = 16
_NW = _NC * _NS
_W = 128
_BLK = _B // _W
_UNITS = _S * _BLK
_UPW = _UNITS // _NW
_L = 16
_NT = _W // _L
_XROWS = 16


def _emb_body(x_hbm, tab_hbm, pe_hbm, out_hbm,
              xbuf, pe_v, hidx0, hidx1, gb0, gb1, tb0, tb1,
              gsem0, gsem1, ssem0, ssem1):
    wid = lax.axis_index("s") * _NC + lax.axis_index("c")
    g0 = wid * _UPW
    s_lo = lax.min((g0 // _BLK) // 8 * 8, _S - _XROWS)

    pltpu.sync_copy(pe_hbm, pe_v)
    pltpu.sync_copy(x_hbm.at[pl.ds(s_lo, _XROWS)], xbuf)

    hidx = (hidx0, hidx1)
    gb = (gb0, gb1)
    tb = (tb0, tb1)
    gsems = (gsem0, gsem1)
    ssems = (ssem0, ssem1)

    iota = lax.iota(jnp.int32, _L)
    rows = [iota + (t * _L) for t in range(_NT)]

    def unit_su(u):
        g = g0 + u
        return g // _BLK, (g % _BLK) * _W

    def xvec(u, t):
        s, b0 = unit_su(u)
        return xbuf[s - s_lo, pl.ds(b0 + t * _L, _L)]

    def fire(u, k):
        for t in range(_NT):
            hidx[k][pl.ds(t * _L, _L)] = lax.shift_right_logical(xvec(u, t), 1)
        pltpu.make_async_copy(tab_hbm.at[hidx[k]], gb[k], gsems[k]).start()

    def wait_gather(k):
        pltpu.make_async_copy(tab_hbm.at[hidx[k]], gb[k], gsems[k]).wait()

    def transpose_add(u, k):
        s, _ = unit_su(u)
        t_ = tb[k]
        g_ = gb[k]
        s_vec = lax.broadcast(s, (_L,))
        par = [lax.shift_left(jnp.bitwise_and(xvec(u, t), 1), 6)
               for t in range(_NT)]

        def d_body(d, carry):
            d_vec = lax.broadcast(d, (_L,))
            pes = plsc.load_gather(pe_v, [s_vec, d_vec])
            for t in range(_NT):
                v = plsc.load_gather(g_, [rows[t], par[t] + d_vec])
                t_[d, pl.ds(t * _L, _L)] = v + pes
            return carry

        lax.fori_loop(0, _D, d_body, 0, unroll=4)

    def store_cp(u, k):
        s, b0 = unit_su(u)
        return pltpu.make_async_copy(
            tb[k], out_hbm.at[s, :, pl.ds(b0, _W)], ssems[k])

    fire(0, 0)
    wait_gather(0)
    fire(1, 1)
    transpose_add(0, 0)
    store_cp(0, 0).start()
    wait_gather(1)
    fire(2, 0)
    transpose_add(1, 1)
    store_cp(1, 1).start()

    def pair_body(p, carry):
        for k in range(2):
            u = 2 * p + k
            wait_gather(k)
            fire(u + 1, 1 - k)
            store_cp(u - 2, k).wait()
            transpose_add(u, k)
            store_cp(u, k).start()
        return carry

    lax.fori_loop(1, _UPW // 2 - 1, pair_body, 0)

    wait_gather(0)
    fire(_UPW - 1, 1)
    store_cp(_UPW - 4, 0).wait()
    transpose_add(_UPW - 2, 0)
    store_cp(_UPW - 2, 0).start()
    wait_gather(1)
    store_cp(_UPW - 3, 1).wait()
    transpose_add(_UPW - 1, 1)
    store_cp(_UPW - 1, 1).start()
    store_cp(_UPW - 2, 0).wait()
    store_cp(_UPW - 1, 1).wait()


def _emb_call(x_t, tab2, pe):
    mesh = plsc.VectorSubcoreMesh(
        core_axis_name="c", subcore_axis_name="s",
        num_cores=_NC, num_subcores=_NS)
    return pl.kernel(
        _emb_body,
        out_type=jax.ShapeDtypeStruct((_S, _D, _B), jnp.float32),
        mesh=mesh,
        compiler_params=pltpu.CompilerParams(needs_layout_passes=False),
        scratch_types=[
            pltpu.VMEM((_XROWS, _B), jnp.int32),
            pltpu.VMEM((_S, _D), jnp.float32),
            pltpu.VMEM((_W,), jnp.int32),
            pltpu.VMEM((_W,), jnp.int32),
            pltpu.VMEM((_W, 2 * _D), jnp.float32),
            pltpu.VMEM((_W, 2 * _D), jnp.float32),
            pltpu.VMEM((_D, _W), jnp.float32),
            pltpu.VMEM((_D, _W), jnp.float32),
            pltpu.SemaphoreType.DMA,
            pltpu.SemaphoreType.DMA,
            pltpu.SemaphoreType.DMA,
            pltpu.SemaphoreType.DMA,
        ],
    )(x_t, tab2, pe)


def kernel(x, table, pe):
    x_t = x.T.astype(jnp.int32)
    tab2 = table.reshape(_V // 2, 2 * _D)
    pe_s = pe[: x.shape[1]]
    out_sdb = _emb_call(x_t, tab2, pe_s)
    return out_sdb.transpose(2, 0, 1)


# speedup vs baseline: 1.2179x; 1.2179x over previous
import jax
import jax.numpy as jnp
from jax import lax
from jax.experimental import pallas as pl
from jax.experimental.pallas import tpu as pltpu
from jax.experimental.pallas import tpu_sc as plsc

_B = 1024
_S = 200
_D = 64
_NC = 2
_NS = 16
_NW = _NC * _NS
_W = 128
_BLK = _B // _W
_UNITS = _S * _BLK
_UPW = _UNITS // _NW
_L = 16
_NT = _W // _L
_VPR = _D // _L
_XROWS = 8


def _emb_body(x_hbm, tab_hbm, pe_hbm, out_hbm,
              xbuf, pe_v, hidx0, hidx1, gb0, gb1,
              gsem0, gsem1, ssem0, ssem1):
    wid = lax.axis_index("s") * _NC + lax.axis_index("c")
    g0 = wid * _UPW
    s_lo = lax.min(g0 // _BLK, _S - _XROWS)

    pltpu.sync_copy(pe_hbm, pe_v)
    pltpu.sync_copy(x_hbm.at[pl.ds(s_lo, _XROWS)], xbuf)

    hidx = (hidx0, hidx1)
    gb = (gb0, gb1)
    gsems = (gsem0, gsem1)
    ssems = (ssem0, ssem1)

    def unit_su(u):
        g = g0 + u
        return g // _BLK, (g % _BLK) * _W

    def fire(u, k):
        s, b0 = unit_su(u)
        for t in range(_NT):
            hidx[k][pl.ds(t * _L, _L)] = xbuf[s - s_lo, pl.ds(b0 + t * _L, _L)]
        pltpu.make_async_copy(tab_hbm.at[hidx[k]], gb[k], gsems[k]).start()

    def wait_gather(k):
        pltpu.make_async_copy(tab_hbm.at[hidx[k]], gb[k], gsems[k]).wait()

    def add_pe(u, k):
        s, _ = unit_su(u)
        g_ = gb[k]
        pev = [pe_v[s, pl.ds(j * _L, _L)] for j in range(_VPR)]

        def r_body(r, carry):
            for j in range(_VPR):
                sl = pl.ds(j * _L, _L)
                g_[r, sl] = g_[r, sl] + pev[j]
            return carry

        lax.fori_loop(0, _W, r_body, 0, unroll=4)

    def store_cp(u, k):
        s, b0 = unit_su(u)
        return pltpu.make_async_copy(
            gb[k], out_hbm.at[s, pl.ds(b0, _W)], ssems[k])

    fire(0, 0)
    wait_gather(0)
    fire(1, 1)
    add_pe(0, 0)
    store_cp(0, 0).start()
    wait_gather(1)
    store_cp(0, 0).wait()
    fire(2, 0)
    add_pe(1, 1)
    store_cp(1, 1).start()

    def pair_body(p, carry):
        for k in range(2):
            u = 2 * p + k
            wait_gather(k)
            store_cp(u - 1, 1 - k).wait()
            fire(u + 1, 1 - k)
            add_pe(u, k)
            store_cp(u, k).start()
        return carry

    lax.fori_loop(1, _UPW // 2 - 1, pair_body, 0)

    wait_gather(0)
    store_cp(_UPW - 3, 1).wait()
    fire(_UPW - 1, 1)
    add_pe(_UPW - 2, 0)
    store_cp(_UPW - 2, 0).start()
    wait_gather(1)
    store_cp(_UPW - 2, 0).wait()
    add_pe(_UPW - 1, 1)
    store_cp(_UPW - 1, 1).start()
    store_cp(_UPW - 1, 1).wait()


def _emb_call(x_t, table, pe):
    mesh = plsc.VectorSubcoreMesh(
        core_axis_name="c", subcore_axis_name="s",
        num_cores=_NC, num_subcores=_NS)
    return pl.kernel(
        _emb_body,
        out_type=jax.ShapeDtypeStruct((_S, _B, _D), jnp.float32),
        mesh=mesh,
        compiler_params=pltpu.CompilerParams(use_tc_tiling_on_sc=False),
        scratch_types=[
            pltpu.VMEM((_XROWS, _B), jnp.int32),
            pltpu.VMEM((_S, _D), jnp.float32),
            pltpu.VMEM((_W,), jnp.int32),
            pltpu.VMEM((_W,), jnp.int32),
            pltpu.VMEM((_W, _D), jnp.float32),
            pltpu.VMEM((_W, _D), jnp.float32),
            pltpu.SemaphoreType.DMA,
            pltpu.SemaphoreType.DMA,
            pltpu.SemaphoreType.DMA,
            pltpu.SemaphoreType.DMA,
        ],
    )(x_t, table, pe)


def kernel(x, table, pe):
    x_t = x.T.astype(jnp.int32)
    pe_s = pe[: x.shape[1]]
    out_sbd = _emb_call(x_t, table, pe_s)
    return out_sbd.transpose(1, 0, 2)
